# baseline (speedup 1.0000x reference)
import functools

import jax
import jax.numpy as jnp
from jax import lax
from jax.experimental import pallas as pl
from jax.experimental.layout import Format, Layout, with_layout_constraint
from jax.experimental.pallas import tpu as pltpu
from jax.experimental.pallas import tpu_sc as plsc

VOCAB = 1000000
EMB = 64
POS = 200
B = 16384
L = 200

N = B * L
NC = 2
NS = 16
NW = NC * NS
TOK_PER_W = N // NW

ROWS_PER_CHUNK = 4
CHUNK_TOK = ROWS_PER_CHUNK * L
NCHUNK = TOK_PER_W // CHUNK_TOK
IDX_MINOR = 100
IDX_ROWS = CHUNK_TOK // IDX_MINOR
NLANE = 16
EMB_VECS = EMB // NLANE


def _body(x_hbm, emb_hbm, pos_hbm, out_hbm, pos_v,
          idx0, idx1, data0, data1, gsem0, gsem1, osem0, osem1):
    wid = lax.axis_index("s") * NC + lax.axis_index("c")
    base_tok = wid * TOK_PER_W
    idx_v = (idx0, idx1)
    data_v = (data0, data1)
    gsem = (gsem0, gsem1)
    osem = (osem0, osem1)

    pltpu.sync_copy(pos_hbm, pos_v)

    def fire_gathers(c, b):
        tok0 = base_tok + c * CHUNK_TOK
        idx_off = pl.multiple_of(tok0 // IDX_MINOR, 8)
        pltpu.sync_copy(x_hbm.at[pl.ds(idx_off, IDX_ROWS)], idx_v[b])
        return [
            pltpu.async_copy(
                emb_hbm.at[idx_v[b].at[j]],
                data_v[b].at[pl.ds(j * IDX_MINOR, IDX_MINOR)],
                gsem[b],
            )
            for j in range(IDX_ROWS)
        ]

    def wait_writeback(c, b):
        tok0 = base_tok + c * CHUNK_TOK
        pltpu.make_async_copy(
            data_v[b], out_hbm.at[pl.ds(tok0, CHUNK_TOK)], osem[b]
        ).wait()

    def add_pos(b):
        def p_body(p, inner):
            buf = data_v[b]
            for v in range(EMB_VECS):
                sl = pl.ds(v * NLANE, NLANE)
                pvec = pos_v[p, sl]
                for rr in range(ROWS_PER_CHUNK):
                    row = rr * L + p
                    buf[row, sl] = buf[row, sl] + pvec
            return inner

        lax.fori_loop(0, L, p_body, 0)

    def start_writeback(c, b):
        tok0 = base_tok + c * CHUNK_TOK
        pltpu.async_copy(
            data_v[b], out_hbm.at[pl.ds(tok0, CHUNK_TOK)], osem[b]
        )

    def process(c, b):
        copies = fire_gathers(c, b)
        for cp in copies:
            cp.wait()
        add_pos(b)
        start_writeback(c, b)

    process(0, 0)
    process(1, 1)

    def pair_body(g, carry):
        for b in range(2):
            c = 2 * g + b
            wait_writeback(c - 2, b)
            process(c, b)
        return carry

    lax.fori_loop(1, NCHUNK // 2, pair_body, 0)
    wait_writeback(NCHUNK - 2, 0)
    wait_writeback(NCHUNK - 1, 1)


def _kernel_impl(x, embedding, position_embedding):
    x2 = x.reshape(N // IDX_MINOR, IDX_MINOR).astype(jnp.int32)
    mesh = plsc.VectorSubcoreMesh(core_axis_name="c", subcore_axis_name="s")
    out = pl.kernel(
        _body,
        out_type=jax.ShapeDtypeStruct((N, EMB), jnp.float32),
        mesh=mesh,
        scratch_types=[
            pltpu.VMEM((POS, EMB), jnp.float32),
            pltpu.VMEM((IDX_ROWS, IDX_MINOR), jnp.int32),
            pltpu.VMEM((IDX_ROWS, IDX_MINOR), jnp.int32),
            pltpu.VMEM((CHUNK_TOK, EMB), jnp.float32),
            pltpu.VMEM((CHUNK_TOK, EMB), jnp.float32),
            pltpu.SemaphoreType.DMA,
            pltpu.SemaphoreType.DMA,
            pltpu.SemaphoreType.DMA,
            pltpu.SemaphoreType.DMA,
        ],
        compiler_params=pltpu.CompilerParams(use_tc_tiling_on_sc=False),
    )(x2, embedding, position_embedding)
    return out.reshape(B, L, EMB)


_kernel_impl.__name__ = "kernel"


@functools.cache
def _jitted_for(dev):
    out_fmt = Format(
        Layout(major_to_minor=(0, 1, 2), tiling=()),
        jax.sharding.SingleDeviceSharding(dev),
    )
    return jax.jit(_kernel_impl, out_shardings=out_fmt)


def kernel(x, embedding, position_embedding):
    try:
        dev = jax.devices("tpu")[0]
    except Exception:
        dev = jax.devices()[0]
    return _jitted_for(dev)(x, embedding, position_embedding)


# speedup vs baseline: 1.5853x; 1.5853x over previous
import functools

import jax
import jax.numpy as jnp
from jax import lax
from jax.experimental import pallas as pl
from jax.experimental.layout import Format, Layout, with_layout_constraint
from jax.experimental.pallas import tpu as pltpu
from jax.experimental.pallas import tpu_sc as plsc

VOCAB = 1000000
EMB = 64
POS = 200
B = 16384
L = 200

N = B * L
NC = 2
NS = 16
NW = NC * NS
TOK_PER_W = N // NW

ROWS_PER_CHUNK = 4
CHUNK_TOK = ROWS_PER_CHUNK * L
NCHUNK = TOK_PER_W // CHUNK_TOK
IDX_MINOR = 100
IDX_ROWS = CHUNK_TOK // IDX_MINOR
NLANE = 16
EMB_VECS = EMB // NLANE


def _body(x_hbm, emb_hbm, pos_hbm, out_hbm, pos_v,
          idx0, idx1, data0, data1, gsem0, gsem1, osem0, osem1):
    wid = lax.axis_index("s") * NC + lax.axis_index("c")
    base_tok = wid * TOK_PER_W
    idx_v = (idx0, idx1)
    data_v = (data0, data1)
    gsem = (gsem0, gsem1)
    osem = (osem0, osem1)

    pltpu.sync_copy(pos_hbm, pos_v)

    def fire_gathers(c, b):
        tok0 = base_tok + c * CHUNK_TOK
        idx_off = pl.multiple_of(tok0 // IDX_MINOR, 8)
        pltpu.sync_copy(x_hbm.at[pl.ds(idx_off, IDX_ROWS)], idx_v[b])
        return [
            pltpu.async_copy(
                emb_hbm.at[idx_v[b].at[j]],
                data_v[b].at[pl.ds(j * IDX_MINOR, IDX_MINOR)],
                gsem[b],
            )
            for j in range(IDX_ROWS)
        ]

    def wait_writeback(c, b):
        tok0 = base_tok + c * CHUNK_TOK
        pltpu.make_async_copy(
            data_v[b],
            out_hbm.at[pl.ds(tok0, CHUNK_TOK), pl.ds(0, EMB)],
            osem[b],
        ).wait()

    def add_pos(b):
        def p_body(p, inner):
            buf = data_v[b]
            for v in range(EMB_VECS):
                sl = pl.ds(v * NLANE, NLANE)
                pvec = pos_v[p, sl]
                for rr in range(ROWS_PER_CHUNK):
                    row = rr * L + p
                    buf[row, sl] = buf[row, sl] + pvec
            return inner

        lax.fori_loop(0, L, p_body, 0)

    def start_writeback(c, b):
        tok0 = base_tok + c * CHUNK_TOK
        pltpu.async_copy(
            data_v[b],
            out_hbm.at[pl.ds(tok0, CHUNK_TOK), pl.ds(0, EMB)],
            osem[b],
        )

    def process(c, b):
        copies = fire_gathers(c, b)
        for cp in copies:
            cp.wait()
        add_pos(b)
        start_writeback(c, b)

    process(0, 0)
    process(1, 1)

    def pair_body(g, carry):
        for b in range(2):
            c = 2 * g + b
            wait_writeback(c - 2, b)
            process(c, b)
        return carry

    lax.fori_loop(1, NCHUNK // 2, pair_body, 0)
    wait_writeback(NCHUNK - 2, 0)
    wait_writeback(NCHUNK - 1, 1)


def _kernel_impl(x, embedding, position_embedding):
    x2 = x.reshape(N // IDX_MINOR, IDX_MINOR).astype(jnp.int32)
    mesh = plsc.VectorSubcoreMesh(core_axis_name="c", subcore_axis_name="s")
    out = pl.kernel(
        _body,
        out_type=jax.ShapeDtypeStruct((N, 2 * EMB), jnp.float32),
        mesh=mesh,
        scratch_types=[
            pltpu.VMEM((POS, EMB), jnp.float32),
            pltpu.VMEM((IDX_ROWS, IDX_MINOR), jnp.int32),
            pltpu.VMEM((IDX_ROWS, IDX_MINOR), jnp.int32),
            pltpu.VMEM((CHUNK_TOK, EMB), jnp.float32),
            pltpu.VMEM((CHUNK_TOK, EMB), jnp.float32),
            pltpu.SemaphoreType.DMA,
            pltpu.SemaphoreType.DMA,
            pltpu.SemaphoreType.DMA,
            pltpu.SemaphoreType.DMA,
        ],
        compiler_params=pltpu.CompilerParams(use_tc_tiling_on_sc=False),
    )(x2, embedding, position_embedding)
    return out.reshape(B, L, 2 * EMB)[:, :, :EMB]


_kernel_impl.__name__ = "kernel"


@functools.cache
def _jitted_for(dev):
    out_fmt = Format(
        Layout(major_to_minor=(0, 1, 2), tiling=((8, 128),)),
        jax.sharding.SingleDeviceSharding(dev),
    )
    return jax.jit(_kernel_impl, out_shardings=out_fmt)


def kernel(x, embedding, position_embedding):
    try:
        dev = jax.devices("tpu")[0]
    except Exception:
        dev = jax.devices()[0]
    return _jitted_for(dev)(x, embedding, position_embedding)


# speedup vs baseline: 1.6018x; 1.0104x over previous
import functools

import jax
import jax.numpy as jnp
from jax import lax
from jax.experimental import pallas as pl
from jax.experimental.layout import Format, Layout, with_layout_constraint
from jax.experimental.pallas import tpu as pltpu
from jax.experimental.pallas import tpu_sc as plsc

VOCAB = 1000000
EMB = 64
POS = 200
B = 16384
L = 200

N = B * L
NC = 2
NS = 16
NW = NC * NS
TOK_PER_W = N // NW

ROWS_PER_CHUNK = 4
CHUNK_TOK = ROWS_PER_CHUNK * L
NCHUNK = TOK_PER_W // CHUNK_TOK
IDX_MINOR = 80
IDX_ROWS = CHUNK_TOK // IDX_MINOR
NLANE = 16
EMB_VECS = EMB // NLANE


def _body(x_hbm, emb_hbm, pos_hbm, out_hbm, pos_v,
          idx0, idx1, data0, data1, gsem0, gsem1, osem0, osem1):
    wid = lax.axis_index("s") * NC + lax.axis_index("c")
    base_tok = wid * TOK_PER_W
    idx_v = (idx0, idx1)
    data_v = (data0, data1)
    gsem = (gsem0, gsem1)
    osem = (osem0, osem1)

    pltpu.sync_copy(pos_hbm, pos_v)

    def fire_gathers(c, b):
        tok0 = base_tok + c * CHUNK_TOK
        idx_off = pl.multiple_of(tok0 // IDX_MINOR, 2)
        pltpu.sync_copy(x_hbm.at[pl.ds(idx_off, IDX_ROWS)], idx_v[b])
        return [
            pltpu.async_copy(
                emb_hbm.at[idx_v[b].at[j]],
                data_v[b].at[pl.ds(j * IDX_MINOR, IDX_MINOR)],
                gsem[b],
            )
            for j in range(IDX_ROWS)
        ]

    def wait_writeback(c, b):
        tok0 = base_tok + c * CHUNK_TOK
        pltpu.make_async_copy(
            data_v[b],
            out_hbm.at[pl.ds(tok0, CHUNK_TOK), pl.ds(0, EMB)],
            osem[b],
        ).wait()

    def add_pos(b):
        def p_body(p, inner):
            buf = data_v[b]
            for v in range(EMB_VECS):
                sl = pl.ds(v * NLANE, NLANE)
                pvec = pos_v[p, sl]
                for rr in range(ROWS_PER_CHUNK):
                    row = rr * L + p
                    buf[row, sl] = buf[row, sl] + pvec
            return inner

        lax.fori_loop(0, L, p_body, 0)

    def start_writeback(c, b):
        tok0 = base_tok + c * CHUNK_TOK
        pltpu.async_copy(
            data_v[b],
            out_hbm.at[pl.ds(tok0, CHUNK_TOK), pl.ds(0, EMB)],
            osem[b],
        )

    def process(c, b):
        copies = fire_gathers(c, b)
        for cp in copies:
            cp.wait()
        add_pos(b)
        start_writeback(c, b)

    process(0, 0)
    process(1, 1)

    def pair_body(g, carry):
        for b in range(2):
            c = 2 * g + b
            wait_writeback(c - 2, b)
            process(c, b)
        return carry

    lax.fori_loop(1, NCHUNK // 2, pair_body, 0)
    wait_writeback(NCHUNK - 2, 0)
    wait_writeback(NCHUNK - 1, 1)


def _kernel_impl(x, embedding, position_embedding):
    x2 = x.reshape(N // IDX_MINOR, IDX_MINOR).astype(jnp.int32)
    mesh = plsc.VectorSubcoreMesh(core_axis_name="c", subcore_axis_name="s")
    out = pl.kernel(
        _body,
        out_type=jax.ShapeDtypeStruct((N, 2 * EMB), jnp.float32),
        mesh=mesh,
        scratch_types=[
            pltpu.VMEM((POS, EMB), jnp.float32),
            pltpu.VMEM((IDX_ROWS, IDX_MINOR), jnp.int32),
            pltpu.VMEM((IDX_ROWS, IDX_MINOR), jnp.int32),
            pltpu.VMEM((CHUNK_TOK, EMB), jnp.float32),
            pltpu.VMEM((CHUNK_TOK, EMB), jnp.float32),
            pltpu.SemaphoreType.DMA,
            pltpu.SemaphoreType.DMA,
            pltpu.SemaphoreType.DMA,
            pltpu.SemaphoreType.DMA,
        ],
        compiler_params=pltpu.CompilerParams(use_tc_tiling_on_sc=False),
    )(x2, embedding, position_embedding)
    return out.reshape(B, L, 2 * EMB)[:, :, :EMB]


_kernel_impl.__name__ = "kernel"


@functools.cache
def _jitted_for(dev):
    out_fmt = Format(
        Layout(major_to_minor=(0, 1, 2), tiling=((8, 128),)),
        jax.sharding.SingleDeviceSharding(dev),
    )
    return jax.jit(_kernel_impl, out_shardings=out_fmt)


def kernel(x, embedding, position_embedding):
    try:
        dev = jax.devices("tpu")[0]
    except Exception:
        dev = jax.devices()[0]
    return _jitted_for(dev)(x, embedding, position_embedding)


# speedup vs baseline: 1.7073x; 1.0659x over previous
import functools

import jax
import jax.numpy as jnp
from jax import lax
from jax.experimental import pallas as pl
from jax.experimental.layout import Format, Layout, with_layout_constraint
from jax.experimental.pallas import tpu as pltpu
from jax.experimental.pallas import tpu_sc as plsc

VOCAB = 1000000
EMB = 64
POS = 200
B = 16384
L = 200

N = B * L
NC = 2
NS = 16
NW = NC * NS
TOK_PER_W = N // NW

ROWS_PER_CHUNK = 4
CHUNK_TOK = ROWS_PER_CHUNK * L
NCHUNK = TOK_PER_W // CHUNK_TOK
IDX_MINOR = 80
IDX_ROWS = CHUNK_TOK // IDX_MINOR
NLANE = 16
EMB_VECS = EMB // NLANE


def _body(x_hbm, emb_hbm, pos_hbm, out_hbm, pos_v,
          idx0, idx1, data0, data1, gsem0, gsem1, osem0, osem1):
    wid = lax.axis_index("s") * NC + lax.axis_index("c")
    base_tok = wid * TOK_PER_W
    idx_v = (idx0, idx1)
    data_v = (data0, data1)
    gsem = (gsem0, gsem1)
    osem = (osem0, osem1)

    pltpu.sync_copy(pos_hbm, pos_v)

    def fire_gathers(c, b):
        tok0 = base_tok + c * CHUNK_TOK
        idx_off = pl.multiple_of(tok0 // IDX_MINOR, 2)
        pltpu.sync_copy(x_hbm.at[pl.ds(idx_off, IDX_ROWS)], idx_v[b])
        return [
            pltpu.async_copy(
                emb_hbm.at[idx_v[b].at[j]],
                data_v[b].at[pl.ds(j * IDX_MINOR, IDX_MINOR)],
                gsem[b],
            )
            for j in range(IDX_ROWS)
        ]

    def wait_writeback(c, b):
        tok0 = base_tok + c * CHUNK_TOK
        pltpu.make_async_copy(
            data_v[b],
            out_hbm.at[pl.ds(tok0, CHUNK_TOK), pl.ds(0, EMB)],
            osem[b],
        ).wait()

    def add_pos(b):
        def p_body(p, inner):
            buf = data_v[b]
            for v in range(EMB_VECS):
                sl = pl.ds(v * NLANE, NLANE)
                pvec = pos_v[p, sl]
                for rr in range(ROWS_PER_CHUNK):
                    row = rr * L + p
                    buf[row, sl] = buf[row, sl] + pvec
            return inner

        lax.fori_loop(0, L, p_body, 0)

    def start_writeback(c, b):
        tok0 = base_tok + c * CHUNK_TOK
        pltpu.async_copy(
            data_v[b],
            out_hbm.at[pl.ds(tok0, CHUNK_TOK), pl.ds(0, EMB)],
            osem[b],
        )

    def drain_gathers(b):
        for j in range(IDX_ROWS):
            pltpu.make_async_copy(
                emb_hbm.at[idx_v[b].at[j]],
                data_v[b].at[pl.ds(j * IDX_MINOR, IDX_MINOR)],
                gsem[b],
            ).wait()

    fire_gathers(0, 0)
    fire_gathers(1, 1)
    drain_gathers(0)
    add_pos(0)
    start_writeback(0, 0)

    def pair_body(g, carry):
        for b in (1, 0):
            c = 2 * g + (1 if b == 1 else 2)
            nb = 1 - b
            wait_writeback(c - 1, nb)
            fire_gathers(c + 1, nb)
            drain_gathers(b)
            add_pos(b)
            start_writeback(c, b)
        return carry

    lax.fori_loop(0, (NCHUNK - 2) // 2, pair_body, 0)
    wait_writeback(NCHUNK - 2, 0)
    drain_gathers(1)
    add_pos(1)
    start_writeback(NCHUNK - 1, 1)
    wait_writeback(NCHUNK - 1, 1)


def _kernel_impl(x, embedding, position_embedding):
    x2 = x.reshape(N // IDX_MINOR, IDX_MINOR).astype(jnp.int32)
    mesh = plsc.VectorSubcoreMesh(core_axis_name="c", subcore_axis_name="s")
    out = pl.kernel(
        _body,
        out_type=jax.ShapeDtypeStruct((N, 2 * EMB), jnp.float32),
        mesh=mesh,
        scratch_types=[
            pltpu.VMEM((POS, EMB), jnp.float32),
            pltpu.VMEM((IDX_ROWS, IDX_MINOR), jnp.int32),
            pltpu.VMEM((IDX_ROWS, IDX_MINOR), jnp.int32),
            pltpu.VMEM((CHUNK_TOK, EMB), jnp.float32),
            pltpu.VMEM((CHUNK_TOK, EMB), jnp.float32),
            pltpu.SemaphoreType.DMA,
            pltpu.SemaphoreType.DMA,
            pltpu.SemaphoreType.DMA,
            pltpu.SemaphoreType.DMA,
        ],
        compiler_params=pltpu.CompilerParams(use_tc_tiling_on_sc=False),
    )(x2, embedding, position_embedding)
    return out.reshape(B, L, 2 * EMB)[:, :, :EMB]


_kernel_impl.__name__ = "kernel"


@functools.cache
def _jitted_for(dev):
    out_fmt = Format(
        Layout(major_to_minor=(0, 1, 2), tiling=((8, 128),)),
        jax.sharding.SingleDeviceSharding(dev),
    )
    return jax.jit(_kernel_impl, out_shardings=out_fmt)


def kernel(x, embedding, position_embedding):
    try:
        dev = jax.devices("tpu")[0]
    except Exception:
        dev = jax.devices()[0]
    return _jitted_for(dev)(x, embedding, position_embedding)
